# baseline (speedup 1.0000x reference)
import functools

import jax
import jax.numpy as jnp
from jax import lax
from jax.experimental import pallas as pl
from jax.experimental.pallas import tpu as pltpu
from jax.experimental.pallas import tpu_sc as plsc

NUM_EMB = 1000000
D = 64
BATCH = 4096
SEQ = 200
SEQ_PAD = 256
B_TOTAL = BATCH * SEQ
NC = 2
NS = 16
NW = NC * NS
ROWS_PW = BATCH // NW
BPW = ROWS_PW * SEQ
R = 2
NSTEP = ROWS_PW // R
NBUF = 2
NGROUP = NSTEP // NBUF


def _embed_body(idx_hbm, table_hbm, out_hbm, idx_v, rows_v, gsems, ssems):
    wid = lax.axis_index("s") * NC + lax.axis_index("c")
    row0 = wid * ROWS_PW
    out0 = wid * BPW
    pltpu.sync_copy(idx_hbm.at[pl.ds(row0, ROWS_PW)], idx_v)

    def fire_gather(step, b):
        for j in range(R):
            pltpu.async_copy(
                table_hbm.at[idx_v.at[step * R + j, pl.ds(0, SEQ)]],
                rows_v.at[b, pl.ds(j * SEQ, SEQ)],
                gsems[b],
            )

    def wait_gather(b):
        pltpu.make_async_copy(
            table_hbm.at[pl.ds(0, R * SEQ)], rows_v.at[b], gsems[b]
        ).wait()

    def fire_store(step, b):
        pltpu.async_copy(
            rows_v.at[b],
            out_hbm.at[pl.ds(out0 + step * R * SEQ, R * SEQ), pl.ds(0, D)],
            ssems[b],
        )

    def wait_store(b):
        pltpu.make_async_copy(
            rows_v.at[b], out_hbm.at[pl.ds(0, R * SEQ), pl.ds(0, D)], ssems[b]
        ).wait()

    for b in range(NBUF):
        fire_gather(b, b)

    def group(g, carry):
        for b in range(NBUF):
            i = g * NBUF + b
            wait_gather(b)
            fire_store(i, b)
            wait_store(b)
            fire_gather(i + NBUF, b)
        return carry

    lax.fori_loop(0, NGROUP - 1, group, 0)

    for b in range(NBUF):
        i = (NGROUP - 1) * NBUF + b
        wait_gather(b)
        fire_store(i, b)
    for b in range(NBUF):
        wait_store(b)


@jax.jit
def _embed(idx_pad, embedding):
    mesh = plsc.VectorSubcoreMesh(
        core_axis_name="c", subcore_axis_name="s", num_cores=NC, num_subcores=NS
    )
    return pl.kernel(
        _embed_body,
        out_type=jax.ShapeDtypeStruct((B_TOTAL, 128), jnp.float32),
        mesh=mesh,
        scratch_types=[
            pltpu.VMEM((ROWS_PW, SEQ_PAD), jnp.int32),
            pltpu.VMEM((NBUF, R * SEQ, D), jnp.float32),
            [pltpu.SemaphoreType.DMA] * NBUF,
            [pltpu.SemaphoreType.DMA] * NBUF,
        ],
        compiler_params=pltpu.CompilerParams(use_tc_tiling_on_sc=False),
    )(idx_pad, embedding)


def kernel(inputs, embedding):
    idx_pad = jnp.pad(inputs, ((0, 0), (0, SEQ_PAD - SEQ)))
    out = _embed(idx_pad, embedding)
    return out[:, :D].reshape(BATCH, SEQ, D)
